# baseline (speedup 1.0000x reference)
import functools
import jax
import jax.numpy as jnp
from jax import lax
from jax.experimental import pallas as pl
from jax.experimental.pallas import tpu as pltpu
from jax.experimental.pallas import tpu_sc as plsc

N = 10000
E = 320000
D = 128
NEG_SLOPE = 0.2

NC = 2
NS = 16
NW = NC * NS
NPW = 320
NPAD = NW * NPW
C = 1600
NCHUNK = E // C


def _tc_body(x_ref, wt_ref, b_ref, att_ref, z_ref, p_ref):
    xb = x_ref[...]
    y = jnp.dot(xb, wt_ref[...], preferred_element_type=jnp.float32) + b_ref[...]
    t = jnp.sum(y * att_ref[...], axis=1, keepdims=True)
    t = jnp.where(t >= 0, t, NEG_SLOPE * t)
    p = jnp.exp(t)
    z_ref[...] = y * p
    p_ref[...] = p


def _node_precompute(x, wt, b, att):
    blk = 1000
    grid = N // blk
    return pl.pallas_call(
        _tc_body,
        grid=(grid,),
        in_specs=[
            pl.BlockSpec((blk, D), lambda i: (i, 0)),
            pl.BlockSpec((D, D), lambda i: (0, 0)),
            pl.BlockSpec((1, D), lambda i: (0, 0)),
            pl.BlockSpec((1, D), lambda i: (0, 0)),
        ],
        out_specs=[
            pl.BlockSpec((blk, D), lambda i: (i, 0)),
            pl.BlockSpec((blk, 1), lambda i: (i, 0)),
        ],
        out_shape=[
            jax.ShapeDtypeStruct((N, D), jnp.float32),
            jax.ShapeDtypeStruct((N, 1), jnp.float32),
        ],
    )(x, wt, b, att)


def _sc_edge_kernel(z_hbm, p_hbm, src_hbm, dst_hbm, xpad_hbm, out_hbm,
                    acc_v, accd_v, p_v,
                    srcb0, dstb0, srcb1, dstb1,
                    srcb2, dstb2, srcb3, dstb3,
                    match_a, match_b,
                    idx0, idx1, rows0, rows1,
                    xb, outb,
                    semc0, semc1, semc2, semc3, semg0, semg1):
    cid = lax.axis_index("c")
    sid = lax.axis_index("s")
    w = sid * NC + cid
    lo = w * NPW
    iota16 = lax.iota(jnp.int32, 16)
    zero16 = jnp.zeros((16,), jnp.float32)

    pltpu.sync_copy(p_hbm, p_v)

    def init_acc(i, carry):
        acc_v[pl.ds(i * 16, 16)] = jnp.full((16,), -jnp.inf, jnp.float32)
        return carry
    lax.fori_loop(0, NPW * D // 16, init_acc, 0)

    def init_d(i, carry):
        accd_v[pl.ds(i * 16, 16)] = zero16
        return carry
    lax.fori_loop(0, NPW // 16, init_d, 0)

    def init_m(i, carry):
        match_a[pl.ds(i * 16, 16)] = jnp.zeros((16,), jnp.int32)
        match_b[pl.ds(i * 16, 16)] = jnp.zeros((16,), jnp.int32)
        return carry
    lax.fori_loop(0, (C + 16) // 16, init_m, 0)

    CBS = ((srcb0, dstb0, semc0), (srcb1, dstb1, semc1),
           (srcb2, dstb2, semc2), (srcb3, dstb3, semc3))
    MS = (match_a, match_b)
    grp_bufs = ((idx0, rows0, semg0), (idx1, rows1, semg1))

    def issue_chunk(ci, cb):
        sb, db, sem = cb
        pltpu.async_copy(src_hbm.at[pl.ds(pl.multiple_of(ci * C, 8), C)], sb, sem)
        pltpu.async_copy(dst_hbm.at[pl.ds(pl.multiple_of(ci * C, 8), C)], db, sem)

    def wait_chunk(cb):
        sb, db, sem = cb
        pltpu.make_async_copy(src_hbm.at[pl.ds(0, C)], sb, sem).wait()
        pltpu.make_async_copy(dst_hbm.at[pl.ds(0, C)], db, sem).wait()

    def filt_chunk(db, mv):
        def filt_one(off, wv):
            dv = db[pl.ds(off, 16)]
            m = (dv >= lo) & (dv < lo + NPW)
            pos = plsc.cumsum(m.astype(jnp.int32)) - 1 + wv
            plsc.store_scatter(mv, [pos], off + iota16, mask=m)
            return wv + plsc.all_reduce_population_count(m)

        def filt(i, wv):
            wv = filt_one(i * 32, wv)
            return filt_one(i * 32 + 16, wv)
        return lax.fori_loop(0, C // 32, filt, jnp.zeros((16,), jnp.int32))

    def issue_grp(g, b, sb, mv):
        ib, rb, sem = grp_bufs[b]
        idx16 = mv[pl.ds(g * 16, 16)]
        src16 = plsc.load_gather(sb, [idx16])
        ib[...] = src16
        pltpu.async_copy(z_hbm.at[ib], rb, sem)

    def process_grp(g, b, db, K, mv):
        ib, rb, sem = grp_bufs[b]
        pltpu.make_async_copy(z_hbm.at[ib], rb, sem).wait()
        gi = g * 16
        idx16 = mv[pl.ds(gi, 16)]
        dst16 = plsc.load_gather(db, [idx16])
        ldst16 = dst16 - lo
        kg = jnp.minimum(K - gi, 16)
        lm = iota16 < kg
        src16 = ib[...]
        p16 = plsc.load_gather(p_v, [src16])
        plsc.addupdate_scatter(accd_v, [ldst16], p16, mask=lm)
        for i in range(16):
            @pl.when(i < kg)
            def _edge():
                rbase = ldst16[i] * D
                for j in range(D // 16):
                    cur = acc_v[pl.ds(rbase + j * 16, 16)]
                    acc_v[pl.ds(rbase + j * 16, 16)] = (
                        jnp.maximum(cur, rb[i, pl.ds(j * 16, 16)]))

    def start_grps(K, sb, mv):
        @pl.when(K > 0)
        def _g0():
            issue_grp(0, 0, sb, mv)

        @pl.when(K > 16)
        def _g1():
            issue_grp(1, 1, sb, mv)

    def drain(K, sb, db, mv):
        def gpair(t, carry):
            g0 = 2 * t
            g1 = g0 + 1
            process_grp(g0, 0, db, K, mv)

            @pl.when((g0 + 2) * 16 < K)
            def _i0():
                issue_grp(g0 + 2, 0, sb, mv)

            @pl.when(g1 * 16 < K)
            def _p1():
                process_grp(g1, 1, db, K, mv)

            @pl.when((g1 + 2) * 16 < K)
            def _i1():
                issue_grp(g1 + 2, 1, sb, mv)
            return carry
        lax.fori_loop(0, (K + 31) // 32, gpair, 0)

    issue_chunk(0, CBS[0])
    issue_chunk(1, CBS[1])
    issue_chunk(2, CBS[2])
    issue_chunk(3, CBS[3])
    wait_chunk(CBS[0])
    K0 = filt_chunk(CBS[0][1], match_a)
    start_grps(K0[0], CBS[0][0], match_a)

    def macro(m, Kc):
        base = m * 4
        for j in range(4):
            c = base + j
            cb_cur = CBS[j]
            mv_cur = MS[j & 1]
            cb_nxt = CBS[(j + 1) & 3]
            mv_nxt = MS[(j + 1) & 1]
            valid = c + 1 < NCHUNK

            @pl.when(valid)
            def _w():
                wait_chunk(cb_nxt)
            Kn = filt_chunk(cb_nxt[1], mv_nxt)
            drain(Kc[0], cb_cur[0], cb_cur[1], mv_cur)
            Kn0 = Kn[0]

            @pl.when(valid & (Kn0 > 0))
            def _g0():
                issue_grp(0, 0, cb_nxt[0], mv_nxt)

            @pl.when(valid & (Kn0 > 16))
            def _g1():
                issue_grp(1, 1, cb_nxt[0], mv_nxt)
            ci = c + 4

            @pl.when(ci < NCHUNK)
            def _pf():
                issue_chunk(ci, CBS[j])
            Kc = Kn
        return Kc
    lax.fori_loop(0, NCHUNK // 4, macro, K0)

    def fin(bi, carry):
        nlo = bi * 16
        dvec = accd_v[pl.ds(nlo, 16)]
        nonempty = dvec > 0
        inv = jnp.where(nonempty, 1.0 / jnp.where(nonempty, dvec, 1.0), 0.0)
        pltpu.sync_copy(xpad_hbm.at[pl.ds(pl.multiple_of(lo + nlo, 8), 16)], xb)
        for f in range(D):
            fidx = jnp.full((16,), f, jnp.int32)
            col = plsc.load_gather(acc_v, [(nlo + iota16) * D + f])
            xcol = plsc.load_gather(xb, [iota16, fidx])
            contrib = jnp.where(nonempty, col * inv, 0.0)
            plsc.store_scatter(outb, [iota16, fidx], contrib + xcol)
        pltpu.sync_copy(outb, out_hbm.at[pl.ds(pl.multiple_of(lo + nlo, 8), 16)])
        return carry
    lax.fori_loop(0, NPW // 16, fin, 0)


@functools.partial(
    pl.kernel,
    out_type=jax.ShapeDtypeStruct((NPAD, D), jnp.float32),
    mesh=plsc.VectorSubcoreMesh(core_axis_name="c", subcore_axis_name="s"),
    compiler_params=pltpu.CompilerParams(needs_layout_passes=False, disable_bounds_checks=True),
    scratch_types=[
        pltpu.VMEM((NPW * D,), jnp.float32),
        pltpu.VMEM((NPW,), jnp.float32),
        pltpu.VMEM((N,), jnp.float32),
        pltpu.VMEM((C,), jnp.int32),
        pltpu.VMEM((C,), jnp.int32),
        pltpu.VMEM((C,), jnp.int32),
        pltpu.VMEM((C,), jnp.int32),
        pltpu.VMEM((C,), jnp.int32),
        pltpu.VMEM((C,), jnp.int32),
        pltpu.VMEM((C,), jnp.int32),
        pltpu.VMEM((C,), jnp.int32),
        pltpu.VMEM((C + 16,), jnp.int32),
        pltpu.VMEM((C + 16,), jnp.int32),
        pltpu.VMEM((16,), jnp.int32),
        pltpu.VMEM((16,), jnp.int32),
        pltpu.VMEM((16, D), jnp.float32),
        pltpu.VMEM((16, D), jnp.float32),
        pltpu.VMEM((16, D), jnp.float32),
        pltpu.VMEM((16, D), jnp.float32),
        pltpu.SemaphoreType.DMA,
        pltpu.SemaphoreType.DMA,
        pltpu.SemaphoreType.DMA,
        pltpu.SemaphoreType.DMA,
        pltpu.SemaphoreType.DMA,
        pltpu.SemaphoreType.DMA,
    ],
)
def _sc_edge(z_hbm, p_hbm, src_hbm, dst_hbm, xpad_hbm, out_hbm,
             acc_v, accd_v, p_v,
             srcb0, dstb0, srcb1, dstb1,
             srcb2, dstb2, srcb3, dstb3,
             match_a, match_b,
             idx0, idx1, rows0, rows1,
             xb, outb,
             semc0, semc1, semc2, semc3, semg0, semg1):
    _sc_edge_kernel(z_hbm, p_hbm, src_hbm, dst_hbm, xpad_hbm, out_hbm,
                    acc_v, accd_v, p_v,
                    srcb0, dstb0, srcb1, dstb1,
                    srcb2, dstb2, srcb3, dstb3,
                    match_a, match_b,
                    idx0, idx1, rows0, rows1,
                    xb, outb,
                    semc0, semc1, semc2, semc3, semg0, semg1)


def kernel(x, edge_index, W_msg, b_msg, att_msg):
    z, p2d = _node_precompute(x, W_msg.T, b_msg.reshape(1, D),
                              att_msg.reshape(1, D))
    p = p2d.reshape(N)
    src = edge_index[0]
    dst = edge_index[1]
    xpad = jnp.concatenate(
        [x, jnp.zeros((NPAD - N, D), jnp.float32)], axis=0)
    zpad = jnp.concatenate(
        [z, jnp.zeros((NPAD - N, D), jnp.float32)], axis=0)
    out = _sc_edge(zpad, p, src, dst, xpad)
    return out[:N]
